# baseline (speedup 1.0000x reference)
import functools
import math

import jax
import jax.numpy as jnp
from jax.experimental import pallas as pl
from jax.experimental.pallas import tpu as pltpu

_B, _CK, _CV, _T, _H, _W = 4, 64, 512, 16, 24, 24
_THW = _T * _H * _W
_HW = _H * _W
_TC = 2304
_NT = _THW // _TC


def _flash_body(qk_ref, mk_ref, mv_ref, out_ref, acc_ref, m_ref, l_ref):
    t = pl.program_id(1)

    @pl.when(t == 0)
    def _init():
        m_ref[...] = jnp.full_like(m_ref, -jnp.inf)
        l_ref[...] = jnp.zeros_like(l_ref)
        acc_ref[...] = jnp.zeros_like(acc_ref)

    k = mk_ref[0]
    q = qk_ref[0]
    v = mv_ref[0]

    a_sq = jnp.sum(k * k, axis=0, keepdims=True)
    ab = jax.lax.dot_general(k, q, (((0,), (0,)), ((), ())),
                             preferred_element_type=jnp.float32)
    s = ab - (a_sq.T * (1.0 / math.sqrt(_CK)))

    m_prev = m_ref[...]
    m_new = jnp.maximum(m_prev, jnp.max(s, axis=0, keepdims=True))
    alpha = jnp.exp(m_prev - m_new)
    p = jnp.exp(s - m_new)

    m_ref[...] = m_new
    l_ref[...] = l_ref[...] * alpha + jnp.sum(p, axis=0, keepdims=True)
    pv = jax.lax.dot_general(v, p, (((1,), (0,)), ((), ())),
                             preferred_element_type=jnp.float32)
    acc_ref[...] = acc_ref[...] * alpha + pv

    @pl.when(t == _NT - 1)
    def _finish():
        out_ref[0] = acc_ref[...] / l_ref[...]


@jax.jit
def kernel(mk, qk, mv, qv):
    b = mk.shape[0]
    mk_f = mk.reshape(b, _CK, _THW)
    mv_f = mv.reshape(b, _CV, _THW)
    qk_f = qk.reshape(b, _CK, _HW) * (2.0 / math.sqrt(_CK))

    mem = pl.pallas_call(
        _flash_body,
        grid=(b, _NT),
        in_specs=[
            pl.BlockSpec((1, _CK, _HW), lambda bb, tt: (bb, 0, 0)),
            pl.BlockSpec((1, _CK, _TC), lambda bb, tt: (bb, 0, tt)),
            pl.BlockSpec((1, _CV, _TC), lambda bb, tt: (bb, 0, tt)),
        ],
        out_specs=pl.BlockSpec((1, _CV, _HW), lambda bb, tt: (bb, 0, 0)),
        out_shape=jax.ShapeDtypeStruct((b, _CV, _HW), jnp.float32),
        scratch_shapes=[
            pltpu.VMEM((_CV, _HW), jnp.float32),
            pltpu.VMEM((1, _HW), jnp.float32),
            pltpu.VMEM((1, _HW), jnp.float32),
        ],
        compiler_params=pltpu.CompilerParams(
            dimension_semantics=("parallel", "arbitrary"),
        ),
    )(qk_f, mk_f, mv_f)

    mem = mem.reshape(b, _CV, _H, _W)
    return jnp.concatenate([mem, qv], axis=1)
